# Initial kernel scaffold; baseline (speedup 1.0000x reference)
import jax
import jax.numpy as jnp
from jax.experimental import pallas as pl


def kernel(enc_outputs_class, enc_outputs_coord_logits_plus_anchors, output_memory, sources_last_element):
    raise NotImplementedError("write your pallas kernel here")


# speedup vs baseline: 1.3485x; 1.3485x over previous
import functools

import jax
import jax.numpy as jnp
from jax import lax
from jax.experimental import pallas as pl
from jax.experimental.pallas import tpu as pltpu
from jax.experimental.pallas import tpu_sc as plsc

NUM_Q = 300
KPAD = 304
CPAD = 320
LANES = 16
MIN32 = -(2 ** 31)


def _bc(n, like):
    return jnp.broadcast_to(jnp.asarray(n, like.dtype), like.shape)


def _srl(x, n):
    return lax.shift_right_logical(x, _bc(n, x))


def _sra(x, n):
    return lax.shift_right_arithmetic(x, _bc(n, x))


def _shl(x, n):
    return lax.shift_left(x, _bc(n, x))


def _tc_cls_max_keys(x):
    B, N, L = x.shape
    BN = 1680

    def body(x_ref, o_ref, p_ref):
        xb = x_ref[...]
        mx = jnp.max(xb, axis=-1)
        kk = lax.bitcast_convert_type(mx, jnp.int32)
        s = kk ^ (lax.shift_right_arithmetic(kk, 31) & jnp.int32(0x7FFFFFFF))
        o_ref[...] = s.reshape(1, 1, 1, BN)
        p_ref[...] = jnp.concatenate(
            [xb, jnp.zeros((1, BN, 128 - L), jnp.float32)], axis=-1)

    keys, classpad = pl.pallas_call(
        body,
        grid=(B, N // BN),
        in_specs=[pl.BlockSpec((1, BN, L), lambda b, n: (b, n, 0))],
        out_specs=[
            pl.BlockSpec((1, 1, 1, BN), lambda b, n: (b, n, 0, 0)),
            pl.BlockSpec((1, BN, 128), lambda b, n: (b, n, 0)),
        ],
        out_shape=[
            jax.ShapeDtypeStruct((B, N // BN, 1, BN), jnp.int32),
            jax.ShapeDtypeStruct((B, N, 128), jnp.float32),
        ],
    )(x)
    return keys.reshape(B, N), classpad.reshape(B * N, 128)


def _sc_topk_gather(cls_keys1d, coord1d, class2d, mem2d):
    BN_total = cls_keys1d.shape[0]
    D = mem2d.shape[-1]
    B = 16
    N = BN_total // B
    NV = N // LANES

    mesh = plsc.VectorSubcoreMesh(core_axis_name="c", subcore_axis_name="s")

    @functools.partial(
        pl.kernel,
        mesh=mesh,
        compiler_params=pltpu.CompilerParams(needs_layout_passes=False),
        out_type=[
            jax.ShapeDtypeStruct((B, 1, KPAD * 4), jnp.float32),
            jax.ShapeDtypeStruct((B, KPAD, D), jnp.float32),
            jax.ShapeDtypeStruct((B, KPAD, 128), jnp.float32),
            jax.ShapeDtypeStruct((B, 1, KPAD * 4), jnp.float32),
        ],
        scratch_types=[
            pltpu.VMEM((N,), jnp.int32),
            pltpu.VMEM((4096,), jnp.int32),
            pltpu.VMEM((CPAD,), jnp.int32),
            pltpu.VMEM((CPAD,), jnp.int32),
            pltpu.VMEM((CPAD,), jnp.int32),
            pltpu.VMEM((CPAD,), jnp.int32),
            pltpu.VMEM((KPAD,), jnp.int32),
            pltpu.VMEM((KPAD,), jnp.int32),
            pltpu.VMEM((N * 4,), jnp.float32),
            pltpu.VMEM((1, KPAD * 4), jnp.float32),
            pltpu.VMEM((1, KPAD * 4), jnp.float32),
            pltpu.VMEM((152, 128), jnp.float32),
            pltpu.VMEM((152, D), jnp.float32),
            pltpu.SemaphoreType.DMA,
            pltpu.SemaphoreType.DMA,
        ],
    )
    def k(cls_hbm, coord_hbm, class_hbm, mem_hbm,
          o_ref, o_tgt, o_log, o_bbox,
          skeys, hist, cs_a, ci_a, cs_b, ci_b,
          selidx_l, selidx_g, slab, coordbuf, bboxbuf, logbuf, membuf,
          sem, sem2):
        wid = lax.axis_index("s") * 2 + lax.axis_index("c")

        @pl.when(wid < B)
        def _():
            b = wid
            lane = lax.broadcasted_iota(jnp.int32, (LANES,), 0)
            lane256 = lane * 256

            pltpu.sync_copy(cls_hbm.at[pl.ds(b * N, N)], skeys)
            slab_dma = pltpu.async_copy(
                coord_hbm.at[pl.ds(b * N * 4, N * 4)], slab, sem2)

            ones16 = jnp.ones((LANES,), jnp.int32)
            zeros16 = jnp.zeros((LANES,), jnp.int32)
            prefix = jnp.int32(0)
            rr = jnp.int32(NUM_Q)

            for sh in (24, 16, 8, 0):
                def clr(i, _):
                    hist[pl.ds(i * LANES, LANES)] = zeros16
                    return 0
                lax.fori_loop(0, 256, clr, 0)

                phi = _srl(prefix, sh + 8) if sh < 24 else None

                def hupd(i, _, _sh=sh, _phi=phi):
                    s = skeys[pl.ds(i * LANES, LANES)]
                    u = s ^ MIN32
                    bkt = _srl(u, _sh) & jnp.int32(255)
                    flat = lane256 + bkt
                    if _phi is None:
                        plsc.addupdate_scatter(hist, [flat], ones16)
                    else:
                        m = _srl(u, _sh + 8) == _phi
                        plsc.addupdate_scatter(hist, [flat], ones16, mask=m)
                    return 0
                lax.fori_loop(0, NV, hupd, 0)

                def bscan(gi, carry, _rr_in=rr):
                    acc_after, j0, sj0 = carry
                    g = 15 - gi
                    h = hist[pl.ds(g * LANES, LANES)]
                    for l in range(1, LANES):
                        h = h + hist[pl.ds(l * 256 + g * LANES, LANES)]
                    incl = lax.rev(jnp.cumsum(lax.rev(h, (0,)), axis=0), (0,))
                    S = (incl - h) + acc_after
                    mlt = S < _rr_in
                    pc = jnp.sum(mlt.astype(jnp.int32))
                    anyb = pc > 0
                    j0l = jnp.minimum(jnp.int32(16) - pc, jnp.int32(15))
                    s_at = jnp.max(
                        S.at[jnp.broadcast_to(j0l, (LANES,))]
                        .get(mode="promise_in_bounds"))
                    j0 = lax.select(anyb, g * LANES + j0l, j0)
                    sj0 = lax.select(anyb, s_at, sj0)
                    acc_after = acc_after + jnp.max(incl)
                    return acc_after, j0, sj0

                _, j0, sj0 = lax.fori_loop(
                    0, 16, bscan,
                    (jnp.int32(0), jnp.int32(0), jnp.int32(0)))
                prefix = prefix | _shl(j0, sh)
                rr = rr - sj0

            ts = prefix ^ MIN32

            def pre(i, _):
                cs_a[pl.ds(i * LANES, LANES)] = jnp.full(
                    (LANES,), MIN32, jnp.int32)
                ci_a[pl.ds(i * LANES, LANES)] = zeros16
                return 0
            lax.fori_loop(0, CPAD // LANES, pre, 0)

            def compact(i, off):
                s = skeys[pl.ds(i * LANES, LANES)]
                m = s >= ts
                mi = m.astype(jnp.int32)
                csum = jnp.cumsum(mi, axis=0)
                pos = off + (csum - mi)
                safe = m & (pos < CPAD)
                plsc.store_scatter(cs_a, [pos], s, mask=safe)
                plsc.store_scatter(ci_a, [pos], i * LANES + lane, mask=safe)
                return off + jnp.max(csum)
            lax.fori_loop(0, NV, compact, jnp.int32(0))

            NCV = CPAD // LANES

            def zcount(i, z):
                s = cs_a[pl.ds(i * LANES, LANES)]
                v = ~(s ^ MIN32)
                bit = v & 1
                return z + (1 - bit)
            z0 = jnp.sum(lax.fori_loop(0, NCV, zcount, zeros16))

            idx15 = jnp.full((LANES,), 15, jnp.int32)

            def bitpass(src_s, src_i, dst_s, dst_i, p, z):
                pn = jnp.minimum(p + 1, jnp.int32(31))

                def mv(i, carry):
                    c0, c1, zn = carry
                    s = src_s[pl.ds(i * LANES, LANES)]
                    ix = src_i[pl.ds(i * LANES, LANES)]
                    v = ~(s ^ MIN32)
                    bit = _srl(v, p) & 1
                    mzi = 1 - bit
                    cs0 = jnp.cumsum(mzi, axis=0)
                    cs1 = jnp.cumsum(bit, axis=0)
                    pos = jnp.where(
                        bit == 0, c0 + (cs0 - mzi), z + c1 + (cs1 - bit))
                    plsc.store_scatter(dst_s, [pos], s)
                    plsc.store_scatter(dst_i, [pos], ix)
                    bn = lax.shift_right_logical(v, pn) & 1
                    t0 = cs0.at[idx15].get(mode="promise_in_bounds")
                    t1 = cs1.at[idx15].get(mode="promise_in_bounds")
                    return c0 + t0, c1 + t1, zn + (1 - bn)

                _, _, zn = lax.fori_loop(
                    0, NCV, mv, (zeros16, zeros16, zeros16))
                return jnp.sum(zn)

            def sortpair(it, z):
                z = bitpass(cs_a, ci_a, cs_b, ci_b, it * 2, z)
                z = bitpass(cs_b, ci_b, cs_a, ci_a, it * 2 + 1, z)
                return z
            lax.fori_loop(0, 16, sortpair, z0)

            def selw(g, _):
                v = ci_a[pl.ds(g * LANES, LANES)]
                selidx_l[pl.ds(g * LANES, LANES)] = v
                selidx_g[pl.ds(g * LANES, LANES)] = v + b * N
                return 0
            lax.fori_loop(0, KPAD // LANES, selw, 0)

            pltpu.async_copy(
                mem_hbm.at[selidx_g.at[pl.ds(0, 152)]], membuf, sem).wait()
            pltpu.sync_copy(membuf, o_tgt.at[b, pl.ds(0, 152)])
            pltpu.async_copy(
                mem_hbm.at[selidx_g.at[pl.ds(152, 152)]], membuf, sem).wait()
            pltpu.sync_copy(membuf, o_tgt.at[b, pl.ds(152, 152)])

            pltpu.async_copy(
                class_hbm.at[selidx_g.at[pl.ds(0, 152)]], logbuf, sem).wait()
            pltpu.sync_copy(logbuf, o_log.at[b, pl.ds(0, 152)])
            pltpu.async_copy(
                class_hbm.at[selidx_g.at[pl.ds(152, 152)]], logbuf, sem).wait()
            pltpu.sync_copy(logbuf, o_log.at[b, pl.ds(152, 152)])

            slab_dma.wait()

            zrow = jnp.zeros((LANES,), jnp.int32)

            def cgather(g, _):
                rows = selidx_l[pl.ds(g * LANES, LANES)]
                pos = g * LANES + lane
                for c in range(4):
                    vals = plsc.load_gather(slab, [rows * 4 + c])
                    plsc.store_scatter(coordbuf, [zrow, pos * 4 + c], vals)
                    sig = 1.0 / (1.0 + jnp.exp(-vals))
                    plsc.store_scatter(bboxbuf, [zrow, pos * 4 + c], sig)
                return 0
            lax.fori_loop(0, KPAD // LANES, cgather, 0)

            pltpu.sync_copy(coordbuf, o_ref.at[b])
            pltpu.sync_copy(bboxbuf, o_bbox.at[b])

    return k(cls_keys1d, coord1d, class2d, mem2d)


def kernel(enc_outputs_class, enc_outputs_coord_logits_plus_anchors,
           output_memory, sources_last_element):
    del sources_last_element
    B, N, L = enc_outputs_class.shape
    D = output_memory.shape[-1]

    cls_keys, classpad = _tc_cls_max_keys(enc_outputs_class)
    refp, tgt, logit, bbox = _sc_topk_gather(
        cls_keys.reshape(B * N),
        enc_outputs_coord_logits_plus_anchors.reshape(B * N * 4),
        classpad,
        output_memory.reshape(B * N, D),
    )
    return (refp.reshape(B, KPAD, 4)[:, :NUM_Q],
            tgt[:, :NUM_Q],
            logit[:, :NUM_Q, :L],
            bbox.reshape(B, KPAD, 4)[:, :NUM_Q])


# speedup vs baseline: 1.4578x; 1.0810x over previous
import functools

import jax
import jax.numpy as jnp
from jax import lax
from jax.experimental import pallas as pl
from jax.experimental.pallas import tpu as pltpu
from jax.experimental.pallas import tpu_sc as plsc

NUM_Q = 300
KPAD = 304
CPAD = 320
LANES = 16
MIN32 = -(2 ** 31)


def _bc(n, like):
    return jnp.broadcast_to(jnp.asarray(n, like.dtype), like.shape)


def _srl(x, n):
    return lax.shift_right_logical(x, _bc(n, x))


def _sra(x, n):
    return lax.shift_right_arithmetic(x, _bc(n, x))


def _shl(x, n):
    return lax.shift_left(x, _bc(n, x))


def _tc_cls_max_keys(x, coords):
    B, N, L = x.shape
    BN = 1680

    def body(x_ref, c_ref, o_ref, p_ref):
        xb = x_ref[...]
        mx = jnp.max(xb, axis=-1)
        kk = lax.bitcast_convert_type(mx, jnp.int32)
        s = kk ^ (lax.shift_right_arithmetic(kk, 31) & jnp.int32(0x7FFFFFFF))
        o_ref[...] = s.reshape(1, 1, 1, BN)
        p_ref[...] = jnp.concatenate(
            [xb, c_ref[...],
             jnp.zeros((1, BN, 128 - L - 4), jnp.float32)], axis=-1)

    keys, classpad = pl.pallas_call(
        body,
        grid=(B, N // BN),
        in_specs=[
            pl.BlockSpec((1, BN, L), lambda b, n: (b, n, 0)),
            pl.BlockSpec((1, BN, 4), lambda b, n: (b, n, 0)),
        ],
        out_specs=[
            pl.BlockSpec((1, 1, 1, BN), lambda b, n: (b, n, 0, 0)),
            pl.BlockSpec((1, BN, 128), lambda b, n: (b, n, 0)),
        ],
        out_shape=[
            jax.ShapeDtypeStruct((B, N // BN, 1, BN), jnp.int32),
            jax.ShapeDtypeStruct((B, N, 128), jnp.float32),
        ],
    )(x, coords)
    return keys.reshape(B, N), classpad.reshape(B * N, 128)


def _sc_topk_gather(cls_keys1d, class2d, mem2d):
    BN_total = cls_keys1d.shape[0]
    D = mem2d.shape[-1]
    B = 16
    N = BN_total // B
    NV = N // LANES

    mesh = plsc.VectorSubcoreMesh(core_axis_name="c", subcore_axis_name="s")

    @functools.partial(
        pl.kernel,
        mesh=mesh,
        compiler_params=pltpu.CompilerParams(needs_layout_passes=False),
        out_type=[
            jax.ShapeDtypeStruct((B, 1, KPAD * 4), jnp.float32),
            jax.ShapeDtypeStruct((B, KPAD, D), jnp.float32),
            jax.ShapeDtypeStruct((B, KPAD, 128), jnp.float32),
            jax.ShapeDtypeStruct((B, 1, KPAD * 4), jnp.float32),
        ],
        scratch_types=[
            pltpu.VMEM((N,), jnp.int32),
            pltpu.VMEM((4096,), jnp.int32),
            pltpu.VMEM((CPAD,), jnp.int32),
            pltpu.VMEM((CPAD,), jnp.int32),
            pltpu.VMEM((CPAD,), jnp.int32),
            pltpu.VMEM((CPAD,), jnp.int32),
            pltpu.VMEM((KPAD,), jnp.int32),
            pltpu.VMEM((KPAD,), jnp.int32),
            pltpu.VMEM((1, KPAD * 4), jnp.float32),
            pltpu.VMEM((1, KPAD * 4), jnp.float32),
            pltpu.VMEM((152, 128), jnp.float32),
            pltpu.VMEM((152, D), jnp.float32),
            pltpu.SemaphoreType.DMA,
        ],
    )
    def k(cls_hbm, class_hbm, mem_hbm,
          o_ref, o_tgt, o_log, o_bbox,
          skeys, hist, cs_a, ci_a, cs_b, ci_b,
          selidx_l, selidx_g, coordbuf, bboxbuf, logbuf, membuf,
          sem):
        wid = lax.axis_index("s") * 2 + lax.axis_index("c")

        @pl.when(wid < B)
        def _():
            b = wid
            lane = lax.broadcasted_iota(jnp.int32, (LANES,), 0)
            lane256 = lane * 256

            pltpu.sync_copy(cls_hbm.at[pl.ds(b * N, N)], skeys)

            ones16 = jnp.ones((LANES,), jnp.int32)
            zeros16 = jnp.zeros((LANES,), jnp.int32)
            prefix = jnp.int32(0)
            rr = jnp.int32(NUM_Q)

            for sh in (24, 16, 8, 0):
                def clr(i, _):
                    hist[pl.ds(i * LANES, LANES)] = zeros16
                    return 0
                lax.fori_loop(0, 256, clr, 0)

                phi = _srl(prefix, sh + 8) if sh < 24 else None

                def hupd(i, _, _sh=sh, _phi=phi):
                    s = skeys[pl.ds(i * LANES, LANES)]
                    u = s ^ MIN32
                    bkt = _srl(u, _sh) & jnp.int32(255)
                    flat = lane256 + bkt
                    if _phi is None:
                        plsc.addupdate_scatter(hist, [flat], ones16)
                    else:
                        m = _srl(u, _sh + 8) == _phi
                        plsc.addupdate_scatter(hist, [flat], ones16, mask=m)
                    return 0
                lax.fori_loop(0, NV, hupd, 0)

                def bscan(gi, carry, _rr_in=rr):
                    acc_after, j0, sj0 = carry
                    g = 15 - gi
                    h = hist[pl.ds(g * LANES, LANES)]
                    for l in range(1, LANES):
                        h = h + hist[pl.ds(l * 256 + g * LANES, LANES)]
                    incl = lax.rev(jnp.cumsum(lax.rev(h, (0,)), axis=0), (0,))
                    S = (incl - h) + acc_after
                    mlt = S < _rr_in
                    pc = jnp.sum(mlt.astype(jnp.int32))
                    anyb = pc > 0
                    j0l = jnp.minimum(jnp.int32(16) - pc, jnp.int32(15))
                    s_at = jnp.max(
                        S.at[jnp.broadcast_to(j0l, (LANES,))]
                        .get(mode="promise_in_bounds"))
                    j0 = lax.select(anyb, g * LANES + j0l, j0)
                    sj0 = lax.select(anyb, s_at, sj0)
                    acc_after = acc_after + jnp.max(incl)
                    return acc_after, j0, sj0

                _, j0, sj0 = lax.fori_loop(
                    0, 16, bscan,
                    (jnp.int32(0), jnp.int32(0), jnp.int32(0)))
                prefix = prefix | _shl(j0, sh)
                rr = rr - sj0

            ts = prefix ^ MIN32

            def pre(i, _):
                cs_a[pl.ds(i * LANES, LANES)] = jnp.full(
                    (LANES,), MIN32, jnp.int32)
                ci_a[pl.ds(i * LANES, LANES)] = zeros16
                return 0
            lax.fori_loop(0, CPAD // LANES, pre, 0)

            def compact(i, off):
                s = skeys[pl.ds(i * LANES, LANES)]
                m = s >= ts
                mi = m.astype(jnp.int32)
                csum = jnp.cumsum(mi, axis=0)
                pos = off + (csum - mi)
                safe = m & (pos < CPAD)
                plsc.store_scatter(cs_a, [pos], s, mask=safe)
                plsc.store_scatter(ci_a, [pos], i * LANES + lane, mask=safe)
                return off + jnp.max(csum)
            lax.fori_loop(0, NV, compact, jnp.int32(0))

            NCV = CPAD // LANES

            def zcount(i, z):
                s = cs_a[pl.ds(i * LANES, LANES)]
                v = ~(s ^ MIN32)
                bit = v & 1
                return z + (1 - bit)
            z0 = jnp.sum(lax.fori_loop(0, NCV, zcount, zeros16))

            idx15 = jnp.full((LANES,), 15, jnp.int32)

            def bitpass(src_s, src_i, dst_s, dst_i, p, z):
                pn = jnp.minimum(p + 1, jnp.int32(31))

                def mv(i, carry):
                    c0, c1, zn = carry
                    s = src_s[pl.ds(i * LANES, LANES)]
                    ix = src_i[pl.ds(i * LANES, LANES)]
                    v = ~(s ^ MIN32)
                    bit = _srl(v, p) & 1
                    mzi = 1 - bit
                    cs0 = jnp.cumsum(mzi, axis=0)
                    cs1 = jnp.cumsum(bit, axis=0)
                    pos = jnp.where(
                        bit == 0, c0 + (cs0 - mzi), z + c1 + (cs1 - bit))
                    plsc.store_scatter(dst_s, [pos], s)
                    plsc.store_scatter(dst_i, [pos], ix)
                    bn = lax.shift_right_logical(v, pn) & 1
                    t0 = cs0.at[idx15].get(mode="promise_in_bounds")
                    t1 = cs1.at[idx15].get(mode="promise_in_bounds")
                    return c0 + t0, c1 + t1, zn + (1 - bn)

                _, _, zn = lax.fori_loop(
                    0, NCV, mv, (zeros16, zeros16, zeros16))
                return jnp.sum(zn)

            def sortpair(it, z):
                z = bitpass(cs_a, ci_a, cs_b, ci_b, it * 2, z)
                z = bitpass(cs_b, ci_b, cs_a, ci_a, it * 2 + 1, z)
                return z
            lax.fori_loop(0, 16, sortpair, z0)

            def selw(g, _):
                v = ci_a[pl.ds(g * LANES, LANES)]
                selidx_l[pl.ds(g * LANES, LANES)] = v
                selidx_g[pl.ds(g * LANES, LANES)] = v + b * N
                return 0
            lax.fori_loop(0, KPAD // LANES, selw, 0)

            pltpu.async_copy(
                mem_hbm.at[selidx_g.at[pl.ds(0, 152)]], membuf, sem).wait()
            pltpu.sync_copy(membuf, o_tgt.at[b, pl.ds(0, 152)])
            pltpu.async_copy(
                mem_hbm.at[selidx_g.at[pl.ds(152, 152)]], membuf, sem).wait()
            pltpu.sync_copy(membuf, o_tgt.at[b, pl.ds(152, 152)])

            zrow = jnp.zeros((LANES,), jnp.int32)
            for off in (0, 152):
                pltpu.async_copy(
                    class_hbm.at[selidx_g.at[pl.ds(off, 152)]],
                    logbuf, sem).wait()
                pltpu.sync_copy(logbuf, o_log.at[b, pl.ds(off, 152)])

                def cext(gg, _, _off=off):
                    p = gg * LANES + lane
                    ps = jnp.minimum(p, 151)
                    ok = p < 152
                    fpos = (_off + p) * 4
                    for c in range(4):
                        cc = jnp.broadcast_to(jnp.int32(80 + c), (LANES,))
                        vals = plsc.load_gather(logbuf, [ps, cc])
                        plsc.store_scatter(
                            coordbuf, [zrow, fpos + c], vals, mask=ok)
                        sig = 1.0 / (1.0 + jnp.exp(-vals))
                        plsc.store_scatter(
                            bboxbuf, [zrow, fpos + c], sig, mask=ok)
                    return 0
                lax.fori_loop(0, 10, cext, 0)

            pltpu.sync_copy(coordbuf, o_ref.at[b])
            pltpu.sync_copy(bboxbuf, o_bbox.at[b])

    return k(cls_keys1d, class2d, mem2d)


def kernel(enc_outputs_class, enc_outputs_coord_logits_plus_anchors,
           output_memory, sources_last_element):
    del sources_last_element
    B, N, L = enc_outputs_class.shape
    D = output_memory.shape[-1]

    cls_keys, classpad = _tc_cls_max_keys(
        enc_outputs_class, enc_outputs_coord_logits_plus_anchors)
    refp, tgt, logit, bbox = _sc_topk_gather(
        cls_keys.reshape(B * N),
        classpad,
        output_memory.reshape(B * N, D),
    )
    return (refp.reshape(B, KPAD, 4)[:, :NUM_Q],
            tgt[:, :NUM_Q],
            logit[:, :NUM_Q, :L],
            bbox.reshape(B, KPAD, 4)[:, :NUM_Q])


# speedup vs baseline: 1.5962x; 1.0949x over previous
import functools

import jax
import jax.numpy as jnp
from jax import lax
from jax.experimental import pallas as pl
from jax.experimental.pallas import tpu as pltpu
from jax.experimental.pallas import tpu_sc as plsc

NUM_Q = 300
KPAD = 304
CPAD = 320
LANES = 16
MIN32 = -(2 ** 31)


def _bc(n, like):
    return jnp.broadcast_to(jnp.asarray(n, like.dtype), like.shape)


def _srl(x, n):
    return lax.shift_right_logical(x, _bc(n, x))


def _sra(x, n):
    return lax.shift_right_arithmetic(x, _bc(n, x))


def _shl(x, n):
    return lax.shift_left(x, _bc(n, x))


def _tc_cls_max_keys(x, coords):
    B, N, L = x.shape
    BN = 4200

    def body(x_ref, c_ref, o_ref, p_ref):
        xb = x_ref[...]
        mx = jnp.max(xb, axis=-1)
        kk = lax.bitcast_convert_type(mx, jnp.int32)
        s = kk ^ (lax.shift_right_arithmetic(kk, 31) & jnp.int32(0x7FFFFFFF))
        o_ref[...] = s.reshape(1, 1, 1, BN)
        p_ref[...] = jnp.concatenate(
            [xb, c_ref[...],
             jnp.zeros((1, BN, 128 - L - 4), jnp.float32)], axis=-1)

    keys, classpad = pl.pallas_call(
        body,
        grid=(B, N // BN),
        in_specs=[
            pl.BlockSpec((1, BN, L), lambda b, n: (b, n, 0)),
            pl.BlockSpec((1, BN, 4), lambda b, n: (b, n, 0)),
        ],
        out_specs=[
            pl.BlockSpec((1, 1, 1, BN), lambda b, n: (b, n, 0, 0)),
            pl.BlockSpec((1, BN, 128), lambda b, n: (b, n, 0)),
        ],
        out_shape=[
            jax.ShapeDtypeStruct((B, N // BN, 1, BN), jnp.int32),
            jax.ShapeDtypeStruct((B, N, 128), jnp.float32),
        ],
    )(x, coords)
    return keys.reshape(B, N), classpad.reshape(B * N, 128)


def _sc_topk_gather(cls_keys1d, class2d, mem2d):
    BN_total = cls_keys1d.shape[0]
    D = mem2d.shape[-1]
    B = 16
    N = BN_total // B
    NV = N // LANES

    mesh = plsc.VectorSubcoreMesh(core_axis_name="c", subcore_axis_name="s")

    @functools.partial(
        pl.kernel,
        mesh=mesh,
        compiler_params=pltpu.CompilerParams(needs_layout_passes=False),
        out_type=[
            jax.ShapeDtypeStruct((B, 1, KPAD * 4), jnp.float32),
            jax.ShapeDtypeStruct((B, KPAD, D), jnp.float32),
            jax.ShapeDtypeStruct((B, KPAD, 128), jnp.float32),
            jax.ShapeDtypeStruct((B, 1, KPAD * 4), jnp.float32),
        ],
        scratch_types=[
            pltpu.VMEM((N,), jnp.int32),
            pltpu.VMEM((4096,), jnp.int32),
            pltpu.VMEM((CPAD,), jnp.int32),
            pltpu.VMEM((CPAD,), jnp.int32),
            pltpu.VMEM((CPAD,), jnp.int32),
            pltpu.VMEM((CPAD,), jnp.int32),
            pltpu.VMEM((KPAD,), jnp.int32),
            pltpu.VMEM((KPAD,), jnp.int32),
            pltpu.VMEM((1, KPAD * 4), jnp.float32),
            pltpu.VMEM((1, KPAD * 4), jnp.float32),
            pltpu.VMEM((152, 128), jnp.float32),
            pltpu.VMEM((152, D), jnp.float32),
            pltpu.SemaphoreType.DMA,
        ],
    )
    def k(cls_hbm, class_hbm, mem_hbm,
          o_ref, o_tgt, o_log, o_bbox,
          skeys, hist, cs_a, ci_a, cs_b, ci_b,
          selidx_l, selidx_g, coordbuf, bboxbuf, logbuf, membuf,
          sem):
        wid = lax.axis_index("s") * 2 + lax.axis_index("c")

        @pl.when(wid < B)
        def _():
            b = wid
            lane = lax.broadcasted_iota(jnp.int32, (LANES,), 0)
            lane256 = lane * 256

            pltpu.sync_copy(cls_hbm.at[pl.ds(b * N, N)], skeys)

            ones16 = jnp.ones((LANES,), jnp.int32)
            zeros16 = jnp.zeros((LANES,), jnp.int32)
            prefix = jnp.int32(0)
            rr = jnp.int32(NUM_Q)

            for sh in (24, 16, 8, 0):
                def clr(i, _):
                    hist[pl.ds(i * LANES, LANES)] = zeros16
                    return 0
                lax.fori_loop(0, 256, clr, 0)

                phi = _srl(prefix, sh + 8) if sh < 24 else None

                def hupd(i, _, _sh=sh, _phi=phi):
                    s = skeys[pl.ds(i * LANES, LANES)]
                    u = s ^ MIN32
                    bkt = _srl(u, _sh) & jnp.int32(255)
                    flat = lane256 + bkt
                    if _phi is None:
                        plsc.addupdate_scatter(hist, [flat], ones16)
                    else:
                        m = _srl(u, _sh + 8) == _phi
                        plsc.addupdate_scatter(hist, [flat], ones16, mask=m)
                    return 0
                lax.fori_loop(0, NV, hupd, 0)

                def bscan(gi, carry, _rr_in=rr):
                    acc_after, j0, sj0 = carry
                    g = 15 - gi
                    h = hist[pl.ds(g * LANES, LANES)]
                    for l in range(1, LANES):
                        h = h + hist[pl.ds(l * 256 + g * LANES, LANES)]
                    incl = lax.rev(jnp.cumsum(lax.rev(h, (0,)), axis=0), (0,))
                    S = (incl - h) + acc_after
                    mlt = S < _rr_in
                    pc = jnp.sum(mlt.astype(jnp.int32))
                    anyb = pc > 0
                    j0l = jnp.minimum(jnp.int32(16) - pc, jnp.int32(15))
                    s_at = jnp.max(
                        S.at[jnp.broadcast_to(j0l, (LANES,))]
                        .get(mode="promise_in_bounds"))
                    j0 = lax.select(anyb, g * LANES + j0l, j0)
                    sj0 = lax.select(anyb, s_at, sj0)
                    acc_after = acc_after + jnp.max(incl)
                    return acc_after, j0, sj0

                _, j0, sj0 = lax.fori_loop(
                    0, 16, bscan,
                    (jnp.int32(0), jnp.int32(0), jnp.int32(0)))
                prefix = prefix | _shl(j0, sh)
                rr = rr - sj0

            ts = prefix ^ MIN32

            def pre(i, _):
                cs_a[pl.ds(i * LANES, LANES)] = jnp.full(
                    (LANES,), MIN32, jnp.int32)
                ci_a[pl.ds(i * LANES, LANES)] = zeros16
                return 0
            lax.fori_loop(0, CPAD // LANES, pre, 0)

            def compact(i, off):
                s = skeys[pl.ds(i * LANES, LANES)]
                m = s >= ts
                mi = m.astype(jnp.int32)
                csum = jnp.cumsum(mi, axis=0)
                pos = off + (csum - mi)
                safe = m & (pos < CPAD)
                plsc.store_scatter(cs_a, [pos], s, mask=safe)
                plsc.store_scatter(ci_a, [pos], i * LANES + lane, mask=safe)
                return off + jnp.max(csum)
            lax.fori_loop(0, NV, compact, jnp.int32(0))

            NCV = CPAD // LANES

            def zcount(i, z):
                s = cs_a[pl.ds(i * LANES, LANES)]
                v = ~(s ^ MIN32)
                bit = v & 1
                return z + (1 - bit)
            z0 = jnp.sum(lax.fori_loop(0, NCV, zcount, zeros16))

            idx15 = jnp.full((LANES,), 15, jnp.int32)

            def bitpass(src_s, src_i, dst_s, dst_i, p, z):
                pn = jnp.minimum(p + 1, jnp.int32(31))

                def mv(i, carry):
                    c0, c1, zn = carry
                    s = src_s[pl.ds(i * LANES, LANES)]
                    ix = src_i[pl.ds(i * LANES, LANES)]
                    v = ~(s ^ MIN32)
                    bit = _srl(v, p) & 1
                    mzi = 1 - bit
                    cs0 = jnp.cumsum(mzi, axis=0)
                    cs1 = jnp.cumsum(bit, axis=0)
                    pos = jnp.where(
                        bit == 0, c0 + (cs0 - mzi), z + c1 + (cs1 - bit))
                    plsc.store_scatter(dst_s, [pos], s)
                    plsc.store_scatter(dst_i, [pos], ix)
                    bn = lax.shift_right_logical(v, pn) & 1
                    t0 = cs0.at[idx15].get(mode="promise_in_bounds")
                    t1 = cs1.at[idx15].get(mode="promise_in_bounds")
                    return c0 + t0, c1 + t1, zn + (1 - bn)

                _, _, zn = lax.fori_loop(
                    0, NCV, mv, (zeros16, zeros16, zeros16))
                return jnp.sum(zn)

            def sortpair(it, z):
                z = bitpass(cs_a, ci_a, cs_b, ci_b, it * 2, z)
                z = bitpass(cs_b, ci_b, cs_a, ci_a, it * 2 + 1, z)
                return z
            lax.fori_loop(0, 16, sortpair, z0)

            def selw(g, _):
                v = ci_a[pl.ds(g * LANES, LANES)]
                selidx_l[pl.ds(g * LANES, LANES)] = v
                selidx_g[pl.ds(g * LANES, LANES)] = v + b * N
                return 0
            lax.fori_loop(0, KPAD // LANES, selw, 0)

            pltpu.async_copy(
                mem_hbm.at[selidx_g.at[pl.ds(0, 152)]], membuf, sem).wait()
            pltpu.sync_copy(membuf, o_tgt.at[b, pl.ds(0, 152)])
            pltpu.async_copy(
                mem_hbm.at[selidx_g.at[pl.ds(152, 152)]], membuf, sem).wait()
            pltpu.sync_copy(membuf, o_tgt.at[b, pl.ds(152, 152)])

            zrow = jnp.zeros((LANES,), jnp.int32)
            for off in (0, 152):
                pltpu.async_copy(
                    class_hbm.at[selidx_g.at[pl.ds(off, 152)]],
                    logbuf, sem).wait()
                pltpu.sync_copy(logbuf, o_log.at[b, pl.ds(off, 152)])

                def cext(gg, _, _off=off):
                    p = gg * LANES + lane
                    ps = jnp.minimum(p, 151)
                    ok = p < 152
                    fpos = (_off + p) * 4
                    for c in range(4):
                        cc = jnp.broadcast_to(jnp.int32(80 + c), (LANES,))
                        vals = plsc.load_gather(logbuf, [ps, cc])
                        plsc.store_scatter(
                            coordbuf, [zrow, fpos + c], vals, mask=ok)
                        sig = 1.0 / (1.0 + jnp.exp(-vals))
                        plsc.store_scatter(
                            bboxbuf, [zrow, fpos + c], sig, mask=ok)
                    return 0
                lax.fori_loop(0, 10, cext, 0)

            pltpu.sync_copy(coordbuf, o_ref.at[b])
            pltpu.sync_copy(bboxbuf, o_bbox.at[b])

    return k(cls_keys1d, class2d, mem2d)


def kernel(enc_outputs_class, enc_outputs_coord_logits_plus_anchors,
           output_memory, sources_last_element):
    del sources_last_element
    B, N, L = enc_outputs_class.shape
    D = output_memory.shape[-1]

    cls_keys, classpad = _tc_cls_max_keys(
        enc_outputs_class, enc_outputs_coord_logits_plus_anchors)
    refp, tgt, logit, bbox = _sc_topk_gather(
        cls_keys.reshape(B * N),
        classpad,
        output_memory.reshape(B * N, D),
    )
    return (refp.reshape(B, KPAD, 4)[:, :NUM_Q],
            tgt[:, :NUM_Q],
            logit[:, :NUM_Q, :L],
            bbox.reshape(B, KPAD, 4)[:, :NUM_Q])


# speedup vs baseline: 1.6517x; 1.0348x over previous
import functools

import jax
import jax.numpy as jnp
from jax import lax
from jax.experimental import pallas as pl
from jax.experimental.pallas import tpu as pltpu
from jax.experimental.pallas import tpu_sc as plsc

NUM_Q = 300
KPAD = 304
CPAD = 320
LANES = 16
MIN32 = -(2 ** 31)


def _bc(n, like):
    return jnp.broadcast_to(jnp.asarray(n, like.dtype), like.shape)


def _srl(x, n):
    return lax.shift_right_logical(x, _bc(n, x))


def _sra(x, n):
    return lax.shift_right_arithmetic(x, _bc(n, x))


def _shl(x, n):
    return lax.shift_left(x, _bc(n, x))


def _tc_cls_max_keys(x, coords):
    B, N, L = x.shape
    BN = 8400

    def body(x_ref, c_ref, o_ref, p_ref):
        xb = x_ref[...]
        mx = jnp.max(xb, axis=-1)
        kk = lax.bitcast_convert_type(mx, jnp.int32)
        s = kk ^ (lax.shift_right_arithmetic(kk, 31) & jnp.int32(0x7FFFFFFF))
        o_ref[...] = s.reshape(1, 1, 1, BN)
        p_ref[...] = jnp.concatenate(
            [xb, c_ref[...],
             jnp.zeros((1, BN, 128 - L - 4), jnp.float32)], axis=-1)

    keys, classpad = pl.pallas_call(
        body,
        grid=(B, N // BN),
        in_specs=[
            pl.BlockSpec((1, BN, L), lambda b, n: (b, n, 0)),
            pl.BlockSpec((1, BN, 4), lambda b, n: (b, n, 0)),
        ],
        out_specs=[
            pl.BlockSpec((1, 1, 1, BN), lambda b, n: (b, n, 0, 0)),
            pl.BlockSpec((1, BN, 128), lambda b, n: (b, n, 0)),
        ],
        out_shape=[
            jax.ShapeDtypeStruct((B, N // BN, 1, BN), jnp.int32),
            jax.ShapeDtypeStruct((B, N, 128), jnp.float32),
        ],
    )(x, coords)
    return keys.reshape(B, N), classpad.reshape(B * N, 128)


def _sc_topk_gather(cls_keys1d, class2d, mem2d):
    BN_total = cls_keys1d.shape[0]
    D = mem2d.shape[-1]
    B = 16
    N = BN_total // B
    NV = N // LANES

    mesh = plsc.VectorSubcoreMesh(core_axis_name="c", subcore_axis_name="s")

    @functools.partial(
        pl.kernel,
        mesh=mesh,
        compiler_params=pltpu.CompilerParams(needs_layout_passes=False),
        out_type=[
            jax.ShapeDtypeStruct((B, 1, KPAD * 4), jnp.float32),
            jax.ShapeDtypeStruct((B, KPAD, D), jnp.float32),
            jax.ShapeDtypeStruct((B, KPAD, 128), jnp.float32),
            jax.ShapeDtypeStruct((B, 1, KPAD * 4), jnp.float32),
        ],
        scratch_types=[
            pltpu.VMEM((N,), jnp.int32),
            pltpu.VMEM((4096,), jnp.int32),
            pltpu.VMEM((CPAD,), jnp.int32),
            pltpu.VMEM((CPAD,), jnp.int32),
            pltpu.VMEM((CPAD,), jnp.int32),
            pltpu.VMEM((CPAD,), jnp.int32),
            pltpu.VMEM((KPAD,), jnp.int32),
            pltpu.VMEM((KPAD,), jnp.int32),
            pltpu.VMEM((1, KPAD * 4), jnp.float32),
            pltpu.VMEM((1, KPAD * 4), jnp.float32),
            pltpu.VMEM((152, 128), jnp.float32),
            pltpu.VMEM((152, D), jnp.float32),
            pltpu.SemaphoreType.DMA,
        ],
    )
    def k(cls_hbm, class_hbm, mem_hbm,
          o_ref, o_tgt, o_log, o_bbox,
          skeys, hist, cs_a, ci_a, cs_b, ci_b,
          selidx_l, selidx_g, coordbuf, bboxbuf, logbuf, membuf,
          sem):
        wid = lax.axis_index("s") * 2 + lax.axis_index("c")

        @pl.when(wid < B)
        def _():
            b = wid
            lane = lax.broadcasted_iota(jnp.int32, (LANES,), 0)
            lane256 = lane * 256

            pltpu.sync_copy(cls_hbm.at[pl.ds(b * N, N)], skeys)

            ones16 = jnp.ones((LANES,), jnp.int32)
            zeros16 = jnp.zeros((LANES,), jnp.int32)
            prefix = jnp.int32(0)
            rr = jnp.int32(NUM_Q)

            for sh in (24, 16, 8, 0):
                def clr(i, _):
                    hist[pl.ds(i * LANES, LANES)] = zeros16
                    return 0
                lax.fori_loop(0, 256, clr, 0)

                phi = _srl(prefix, sh + 8) if sh < 24 else None

                def hupd(i, _, _sh=sh, _phi=phi):
                    s = skeys[pl.ds(i * LANES, LANES)]
                    u = s ^ MIN32
                    bkt = _srl(u, _sh) & jnp.int32(255)
                    flat = lane256 + bkt
                    if _phi is None:
                        plsc.addupdate_scatter(hist, [flat], ones16)
                    else:
                        m = _srl(u, _sh + 8) == _phi
                        plsc.addupdate_scatter(hist, [flat], ones16, mask=m)
                    return 0
                lax.fori_loop(0, NV, hupd, 0)

                def bscan(gi, carry, _rr_in=rr):
                    acc_after, j0, sj0 = carry
                    g = 15 - gi
                    h = hist[pl.ds(g * LANES, LANES)]
                    for l in range(1, LANES):
                        h = h + hist[pl.ds(l * 256 + g * LANES, LANES)]
                    incl = lax.rev(jnp.cumsum(lax.rev(h, (0,)), axis=0), (0,))
                    S = (incl - h) + acc_after
                    mlt = S < _rr_in
                    pc = jnp.sum(mlt.astype(jnp.int32))
                    anyb = pc > 0
                    j0l = jnp.minimum(jnp.int32(16) - pc, jnp.int32(15))
                    s_at = jnp.max(
                        S.at[jnp.broadcast_to(j0l, (LANES,))]
                        .get(mode="promise_in_bounds"))
                    j0 = lax.select(anyb, g * LANES + j0l, j0)
                    sj0 = lax.select(anyb, s_at, sj0)
                    acc_after = acc_after + jnp.max(incl)
                    return acc_after, j0, sj0

                _, j0, sj0 = lax.fori_loop(
                    0, 16, bscan,
                    (jnp.int32(0), jnp.int32(0), jnp.int32(0)))
                prefix = prefix | _shl(j0, sh)
                rr = rr - sj0

            ts = prefix ^ MIN32

            def pre(i, _):
                cs_a[pl.ds(i * LANES, LANES)] = jnp.full(
                    (LANES,), MIN32, jnp.int32)
                ci_a[pl.ds(i * LANES, LANES)] = zeros16
                return 0
            lax.fori_loop(0, CPAD // LANES, pre, 0)

            def compact(i, off):
                s = skeys[pl.ds(i * LANES, LANES)]
                m = s >= ts
                mi = m.astype(jnp.int32)
                csum = jnp.cumsum(mi, axis=0)
                pos = off + (csum - mi)
                safe = m & (pos < CPAD)
                plsc.store_scatter(cs_a, [pos], s, mask=safe)
                plsc.store_scatter(ci_a, [pos], i * LANES + lane, mask=safe)
                return off + jnp.max(csum)
            lax.fori_loop(0, NV, compact, jnp.int32(0))

            NCV = CPAD // LANES

            def zcount(i, z):
                s = cs_a[pl.ds(i * LANES, LANES)]
                v = ~(s ^ MIN32)
                bit = v & 1
                return z + (1 - bit)
            z0 = jnp.sum(lax.fori_loop(0, NCV, zcount, zeros16))

            idx15 = jnp.full((LANES,), 15, jnp.int32)

            def bitpass(src_s, src_i, dst_s, dst_i, p, z):
                pn = jnp.minimum(p + 1, jnp.int32(31))

                def mv(i, carry):
                    c0, c1, zn = carry
                    s = src_s[pl.ds(i * LANES, LANES)]
                    ix = src_i[pl.ds(i * LANES, LANES)]
                    v = ~(s ^ MIN32)
                    bit = _srl(v, p) & 1
                    mzi = 1 - bit
                    cs0 = jnp.cumsum(mzi, axis=0)
                    cs1 = jnp.cumsum(bit, axis=0)
                    pos = jnp.where(
                        bit == 0, c0 + (cs0 - mzi), z + c1 + (cs1 - bit))
                    plsc.store_scatter(dst_s, [pos], s)
                    plsc.store_scatter(dst_i, [pos], ix)
                    bn = lax.shift_right_logical(v, pn) & 1
                    t0 = cs0.at[idx15].get(mode="promise_in_bounds")
                    t1 = cs1.at[idx15].get(mode="promise_in_bounds")
                    return c0 + t0, c1 + t1, zn + (1 - bn)

                _, _, zn = lax.fori_loop(
                    0, NCV, mv, (zeros16, zeros16, zeros16))
                return jnp.sum(zn)

            def sortpair(it, z):
                z = bitpass(cs_a, ci_a, cs_b, ci_b, it * 2, z)
                z = bitpass(cs_b, ci_b, cs_a, ci_a, it * 2 + 1, z)
                return z
            lax.fori_loop(0, 16, sortpair, z0)

            def selw(g, _):
                v = ci_a[pl.ds(g * LANES, LANES)]
                selidx_l[pl.ds(g * LANES, LANES)] = v
                selidx_g[pl.ds(g * LANES, LANES)] = v + b * N
                return 0
            lax.fori_loop(0, KPAD // LANES, selw, 0)

            pltpu.async_copy(
                mem_hbm.at[selidx_g.at[pl.ds(0, 152)]], membuf, sem).wait()
            pltpu.sync_copy(membuf, o_tgt.at[b, pl.ds(0, 152)])
            pltpu.async_copy(
                mem_hbm.at[selidx_g.at[pl.ds(152, 152)]], membuf, sem).wait()
            pltpu.sync_copy(membuf, o_tgt.at[b, pl.ds(152, 152)])

            zrow = jnp.zeros((LANES,), jnp.int32)
            for off in (0, 152):
                pltpu.async_copy(
                    class_hbm.at[selidx_g.at[pl.ds(off, 152)]],
                    logbuf, sem).wait()
                pltpu.sync_copy(logbuf, o_log.at[b, pl.ds(off, 152)])

                def cext(gg, _, _off=off):
                    p = gg * LANES + lane
                    ps = jnp.minimum(p, 151)
                    ok = p < 152
                    fpos = (_off + p) * 4
                    for c in range(4):
                        cc = jnp.broadcast_to(jnp.int32(80 + c), (LANES,))
                        vals = plsc.load_gather(logbuf, [ps, cc])
                        plsc.store_scatter(
                            coordbuf, [zrow, fpos + c], vals, mask=ok)
                        sig = 1.0 / (1.0 + jnp.exp(-vals))
                        plsc.store_scatter(
                            bboxbuf, [zrow, fpos + c], sig, mask=ok)
                    return 0
                lax.fori_loop(0, 10, cext, 0)

            pltpu.sync_copy(coordbuf, o_ref.at[b])
            pltpu.sync_copy(bboxbuf, o_bbox.at[b])

    return k(cls_keys1d, class2d, mem2d)


def kernel(enc_outputs_class, enc_outputs_coord_logits_plus_anchors,
           output_memory, sources_last_element):
    del sources_last_element
    B, N, L = enc_outputs_class.shape
    D = output_memory.shape[-1]

    cls_keys, classpad = _tc_cls_max_keys(
        enc_outputs_class, enc_outputs_coord_logits_plus_anchors)
    refp, tgt, logit, bbox = _sc_topk_gather(
        cls_keys.reshape(B * N),
        classpad,
        output_memory.reshape(B * N, D),
    )
    return (refp.reshape(B, KPAD, 4)[:, :NUM_Q],
            tgt[:, :NUM_Q],
            logit[:, :NUM_Q, :L],
            bbox.reshape(B, KPAD, 4)[:, :NUM_Q])
